# baseline (speedup 1.0000x reference)
import functools

import jax
import jax.numpy as jnp
from jax import lax
from jax.experimental import pallas as pl
from jax.experimental.pallas import tpu as pltpu
from jax.experimental.pallas import tpu_sc as plsc

BATCH = 4096
SEQ = 200
D = 64
VOCAB_PAIRS = 500000
SCALE = 8.0

_info = plsc.get_sparse_core_info()
NC, NS, L = _info.num_cores, _info.num_subcores, _info.num_lanes
NW = NC * NS

BCOLS = BATCH // NW
N_CHUNKS = SEQ
NB = 4
LA = 2
NLG = BCOLS // L


def _emb_body(xt_hbm, t2_hbm, out_hbm, idx_slab, idx2, rows2, sel, *sems):
    gsem, ssem = sems[:NB], sems[NB:]
    wid = lax.axis_index("s") * NC + lax.axis_index("c")
    b0 = wid * BCOLS

    pltpu.sync_copy(xt_hbm.at[:, pl.ds(b0, BCOLS)], idx_slab)

    def gather_ops(b):
        return t2_hbm.at[idx2.at[b]], rows2.at[b]

    def start_gather(j, b):
        def lg(i, c):
            v = idx_slab[j, pl.ds(i * L, L)]
            idx2[b, pl.ds(i * L, L)] = lax.shift_right_logical(v, 1)
            return c

        lax.fori_loop(0, NLG, lg, 0)
        src, dst = gather_ops(b)
        pltpu.async_copy(src, dst, gsem[b])

    def wait_gather(b):
        src, dst = gather_ops(b)
        pltpu.make_async_copy(src, dst, gsem[b]).wait()

    def scatter_ops(j, b):
        return sel.at[b], out_hbm.at[j, :, pl.ds(b0, BCOLS)]

    def start_scatter(j, b):
        src, dst = scatter_ops(j, b)
        pltpu.async_copy(src, dst, ssem[b])

    def wait_scatter(j, b):
        src, dst = scatter_ops(j, b)
        pltpu.make_async_copy(src, dst, ssem[b]).wait()

    for b in range(LA):
        start_gather(b, b)

    def group(g, carry):
        for b in range(NB):
            j = g * NB + b
            jf = j + LA
            bf = (b + LA) % NB

            @pl.when(jnp.logical_and(jf < N_CHUNKS, jf >= NB))
            def _():
                wait_scatter(jf - NB, bf)

            @pl.when(jf < N_CHUNKS)
            def _():
                start_gather(jf, bf)

            wait_gather(b)

            def lg(i, carry2):
                base = i * L
                par = lax.bitwise_and(idx_slab[j, pl.ds(base, L)], 1)
                col0 = par * D
                lrow = lax.iota(jnp.int32, L) + base
                for d in range(D):
                    v = plsc.load_gather(rows2.at[b], [lrow, col0 + d])
                    sel[b, d, pl.ds(base, L)] = v * SCALE
                return carry2

            lax.fori_loop(0, NLG, lg, 0)
            start_scatter(j, b)
        return carry

    lax.fori_loop(0, N_CHUNKS // NB, group, 0)

    for b in range(NB):
        wait_scatter(N_CHUNKS - NB + b, b)


_emb_kernel = functools.partial(
    pl.kernel,
    out_type=jax.ShapeDtypeStruct((SEQ, D, BATCH), jnp.float32),
    mesh=plsc.VectorSubcoreMesh(core_axis_name="c", subcore_axis_name="s"),
    compiler_params=pltpu.CompilerParams(use_tc_tiling_on_sc=True,
                                         needs_layout_passes=False),
    scratch_types=(
        [pltpu.VMEM((SEQ, BCOLS), jnp.int32),
         pltpu.VMEM((NB, BCOLS), jnp.int32),
         pltpu.VMEM((NB, BCOLS, 2 * D), jnp.float32),
         pltpu.VMEM((NB, D, BCOLS), jnp.float32)]
        + [pltpu.SemaphoreType.DMA] * (2 * NB)
    ),
)(_emb_body)


def kernel(x, table):
    xt = x.T
    t2 = table.reshape(VOCAB_PAIRS, 2 * D)
    out_t = _emb_kernel(xt, t2)
    return jnp.transpose(out_t, (2, 0, 1))


# speedup vs baseline: 1.7597x; 1.7597x over previous
import functools

import jax
import jax.numpy as jnp
from jax import lax
from jax.experimental import pallas as pl
from jax.experimental.pallas import tpu as pltpu
from jax.experimental.pallas import tpu_sc as plsc

BATCH = 4096
SEQ = 200
D = 64
SCALE = 8.0

_info = plsc.get_sparse_core_info()
NC, NS, L = _info.num_cores, _info.num_subcores, _info.num_lanes
NW = NC * NS

ROWS_W = BATCH // NW
CLEN = (104, 96)
COFF = (0, 104)
CPR = 2
N_CHUNKS = ROWS_W * CPR
NB = 4
LA = 2


def _emb_body(x_hbm, t3_hbm, out_hbm, i0, i1, i2, i3, rows, sel, *sems):
    idxr = (i0, i1, i2, i3)
    gsem, ssem = sems[:NB], sems[NB:]
    wid = lax.axis_index("s") * NC + lax.axis_index("c")
    bbase = wid * ROWS_W
    fbase = bbase * SEQ

    def load_idx(j, b, h):
        br = j // CPR
        pltpu.sync_copy(x_hbm.at[pl.ds(fbase + br * SEQ + COFF[h], CLEN[h])],
                        idxr[b])

    def gather_ops(j, b, h):
        src = t3_hbm.at[idxr[b]]
        dst = rows.at[b, pl.ds(0, CLEN[h])]
        return src, dst

    def scatter_ops(j, b, h):
        br = j // CPR
        src = sel.at[b, pl.ds(0, CLEN[h])]
        dst = out_hbm.at[bbase + br, pl.ds(COFF[h], CLEN[h]), :]
        return src, dst

    def start_gather(j, b, h):
        load_idx(j, b, h)
        src, dst = gather_ops(j, b, h)
        pltpu.async_copy(src, dst, gsem[b])

    def wait_gather(j, b, h):
        src, dst = gather_ops(j, b, h)
        pltpu.make_async_copy(src, dst, gsem[b]).wait()

    def start_scatter(j, b, h):
        src, dst = scatter_ops(j, b, h)
        pltpu.async_copy(src, dst, ssem[b])

    def wait_scatter(j, b, h):
        src, dst = scatter_ops(j, b, h)
        pltpu.make_async_copy(src, dst, ssem[b]).wait()

    for b in range(LA):
        start_gather(b, b, b % CPR)

    def group(g, carry):
        for b in range(NB):
            j = g * NB + b
            h = b % CPR
            jf = j + LA
            bf = (b + LA) % NB
            hf = bf % CPR

            @pl.when(jnp.logical_and(jf < N_CHUNKS, jf >= NB))
            def _():
                wait_scatter(jf - NB, bf, hf)

            @pl.when(jf < N_CHUNKS)
            def _():
                start_gather(jf, bf, hf)

            wait_gather(j, b, h)

            def row(i, c2):
                r = i * 2
                for rr in range(2):
                    for c in range(D // L):
                        sel[b, r + rr, pl.ds(c * L, L)] = (
                            rows[b, r + rr, pl.ds(c * L, L)] * SCALE)
                return c2

            lax.fori_loop(0, CLEN[h] // 2, row, 0)
            start_scatter(j, b, h)
        return carry

    lax.fori_loop(0, N_CHUNKS // NB, group, 0)

    for b in range(NB):
        wait_scatter(N_CHUNKS - NB + b, b, b % CPR)


_emb_kernel = functools.partial(
    pl.kernel,
    out_type=jax.ShapeDtypeStruct((BATCH, SEQ, D), jnp.float32),
    mesh=plsc.VectorSubcoreMesh(core_axis_name="c", subcore_axis_name="s"),
    compiler_params=pltpu.CompilerParams(use_tc_tiling_on_sc=True,
                                         needs_layout_passes=False),
    scratch_types=(
        [pltpu.VMEM((CLEN[b % CPR],), jnp.int32) for b in range(NB)]
        + [pltpu.VMEM((NB, CLEN[0], 2 * D), jnp.float32),
           pltpu.VMEM((NB, CLEN[0], D), jnp.float32)]
        + [pltpu.SemaphoreType.DMA] * (2 * NB)
    ),
)(_emb_body)


def kernel(x, table):
    xf = x.reshape(BATCH * SEQ)
    t3 = jnp.pad(table, ((0, 0), (0, D)))
    return _emb_kernel(xf, t3)
